# Initial kernel scaffold; baseline (speedup 1.0000x reference)
import jax
import jax.numpy as jnp
from jax.experimental import pallas as pl


def kernel(x, edge_index, W1, b1, W2, b2):
    raise NotImplementedError("write your pallas kernel here")


# speedup vs baseline: 13.6355x; 13.6355x over previous
import functools

import jax
import jax.numpy as jnp
from jax import lax
from jax.experimental import pallas as pl
from jax.experimental.pallas import tpu as pltpu
from jax.experimental.pallas import tpu_sc as plsc

N = 10000
E = 320000
D = 128
NP = 10240
NC = 2
NS = 16
NW = NC * NS
KE = 128
CE = 80
EP = NW * CE * KE
TPT = NP // NS
RB = 1024
GRID = NP // RB

_mesh = plsc.VectorSubcoreMesh(core_axis_name="c", subcore_axis_name="s")


@functools.partial(
    pl.kernel,
    out_type=jax.ShapeDtypeStruct((NC, NP), jnp.float32),
    mesh=_mesh,
    scratch_types=[
        pltpu.VMEM((CE, KE), jnp.int32),
        pltpu.VMEM((KE,), jnp.float32),
        pltpu.VMEM_SHARED((NP,), jnp.float32),
    ],
)
def _sc_degree(dst_hbm, ones_hbm, z1_hbm, deg_hbm, dstv, onesv, acc):
    cid = lax.axis_index("c")
    sid = lax.axis_index("s")
    wid = cid * NS + sid
    pltpu.sync_copy(dst_hbm.at[wid], dstv)
    pltpu.sync_copy(ones_hbm, onesv)
    pltpu.sync_copy(z1_hbm, acc.at[pl.ds(sid * TPT, TPT)])
    plsc.subcore_barrier()

    @pl.loop(0, CE)
    def _(c):
        pltpu.sync_copy(onesv, acc.at[dstv.at[c]], add=True)

    plsc.subcore_barrier()
    pltpu.sync_copy(acc.at[pl.ds(sid * TPT, TPT)],
                    deg_hbm.at[cid, pl.ds(sid * TPT, TPT)])


@functools.partial(
    pl.kernel,
    out_type=jax.ShapeDtypeStruct((NC, NP, D), jnp.float32),
    mesh=_mesh,
    scratch_types=[
        pltpu.VMEM((CE, KE), jnp.int32),
        pltpu.VMEM((CE, KE), jnp.int32),
        pltpu.VMEM((KE, D), jnp.float32),
        pltpu.VMEM_SHARED((NP, D), jnp.float32),
        pltpu.SemaphoreType.DMA,
    ],
)
def _sc_agg_rows(g_hbm, src_hbm, dst_hbm, z2_hbm, out_hbm,
                 srcv, dstv, buf, acc, sem):
    cid = lax.axis_index("c")
    sid = lax.axis_index("s")
    wid = cid * NS + sid
    pltpu.sync_copy(src_hbm.at[wid], srcv)
    pltpu.sync_copy(dst_hbm.at[wid], dstv)
    pltpu.sync_copy(z2_hbm, acc.at[pl.ds(sid * TPT, TPT)])
    plsc.subcore_barrier()

    @pl.loop(0, CE)
    def _(c):
        pltpu.async_copy(g_hbm.at[srcv.at[c]], buf, sem).wait()
        pltpu.sync_copy(buf, acc.at[dstv.at[c]], add=True)

    plsc.subcore_barrier()
    pltpu.sync_copy(acc.at[pl.ds(sid * TPT, TPT)],
                    out_hbm.at[cid, pl.ds(sid * TPT, TPT)])


@functools.partial(
    pl.kernel,
    out_type=jax.ShapeDtypeStruct((NP,), jnp.float32),
    mesh=_mesh,
    scratch_types=[
        pltpu.VMEM((CE, KE), jnp.int32),
        pltpu.VMEM((CE, KE), jnp.int32),
        pltpu.VMEM((KE,), jnp.float32),
        pltpu.VMEM((TPT,), jnp.float32),
        pltpu.VMEM((TPT,), jnp.float32),
        pltpu.VMEM((TPT,), jnp.float32),
        pltpu.VMEM((TPT,), jnp.float32),
        pltpu.VMEM((16,), jnp.float32),
        pltpu.VMEM_SHARED((NP,), jnp.float32),
        pltpu.SemaphoreType.DMA,
    ],
)
def _sc_agg_scalar(u_hbm, dis_hbm, src_hbm, dst_hbm, z1_hbm, b2_hbm, out_hbm,
                   srcv, dstv, vals, accv, disv, uv, outv, b2v, acc, sem):
    cid = lax.axis_index("c")
    sid = lax.axis_index("s")
    pltpu.sync_copy(b2_hbm, b2v)
    pltpu.sync_copy(z1_hbm, acc.at[pl.ds(sid * TPT, TPT)])
    plsc.subcore_barrier()

    for si in (sid, NS + sid):
        pltpu.sync_copy(src_hbm.at[si], srcv)
        pltpu.sync_copy(dst_hbm.at[si], dstv)

        @pl.loop(0, CE)
        def _(c):
            pltpu.async_copy(u_hbm.at[srcv.at[c]], vals, sem).wait()
            pltpu.sync_copy(vals, acc.at[dstv.at[c]], add=True)

    plsc.subcore_barrier()
    base = sid * TPT
    pltpu.sync_copy(acc.at[pl.ds(base, TPT)], accv)
    pltpu.sync_copy(dis_hbm.at[pl.ds(base, TPT)], disv)
    pltpu.sync_copy(u_hbm.at[pl.ds(base, TPT)], uv)
    b2 = b2v[...]

    @pl.loop(0, TPT // 16)
    def _(i):
        a = accv[pl.ds(i * 16, 16)]
        uu = uv[pl.ds(i * 16, 16)]
        dd = disv[pl.ds(i * 16, 16)]
        outv[pl.ds(i * 16, 16)] = dd * (a + uu) + b2

    @pl.when(cid == 0)
    def _():
        pltpu.sync_copy(outv, out_hbm.at[pl.ds(base, TPT)])


def _tc_mm1(x, w1, degp):
    def body(xr, wr, dr, gr, disr):
        h = jnp.dot(xr[...], wr[...], preferred_element_type=jnp.float32)
        dis = lax.rsqrt(dr[0] + dr[1] + 1.0)
        disr[...] = dis
        gr[...] = h * dis

    return pl.pallas_call(
        body,
        grid=(GRID,),
        in_specs=[
            pl.BlockSpec((RB, D), lambda i: (i, 0)),
            pl.BlockSpec((D, D), lambda i: (0, 0)),
            pl.BlockSpec((NC, RB, 1), lambda i: (0, i, 0)),
        ],
        out_specs=[
            pl.BlockSpec((RB, D), lambda i: (i, 0)),
            pl.BlockSpec((RB, 1), lambda i: (i, 0)),
        ],
        out_shape=[
            jax.ShapeDtypeStruct((NP, D), jnp.float32),
            jax.ShapeDtypeStruct((NP, 1), jnp.float32),
        ],
    )(x, w1, degp)


def _tc_mm2(p, g, dis, b1, w2):
    def body(pr, gr, disr, br, wr, ur):
        s = (pr[0] + pr[1] + gr[...]) * disr[...] + br[...]
        h1 = jnp.maximum(s, 0.0)
        t = jnp.dot(h1, wr[...], preferred_element_type=jnp.float32)
        ur[...] = t * disr[...]

    return pl.pallas_call(
        body,
        grid=(GRID,),
        in_specs=[
            pl.BlockSpec((NC, RB, D), lambda i: (0, i, 0)),
            pl.BlockSpec((RB, D), lambda i: (i, 0)),
            pl.BlockSpec((RB, 1), lambda i: (i, 0)),
            pl.BlockSpec((1, D), lambda i: (0, 0)),
            pl.BlockSpec((D, 1), lambda i: (0, 0)),
        ],
        out_specs=pl.BlockSpec((RB, 1), lambda i: (i, 0)),
        out_shape=jax.ShapeDtypeStruct((NP, 1), jnp.float32),
    )(p, g, dis, b1, w2)


@jax.jit
def kernel(x, edge_index, W1, b1, W2, b2):
    pad = jnp.zeros((EP - E,), jnp.int32)
    src = jnp.concatenate([edge_index[0], pad]).reshape(NW, CE, KE)
    dst = jnp.concatenate([edge_index[1], pad + N]).reshape(NW, CE, KE)
    xp = jnp.pad(x, ((0, NP - N), (0, 0)))
    ones_k = jnp.ones((KE,), jnp.float32)
    z1 = jnp.zeros((TPT,), jnp.float32)
    z2 = jnp.zeros((TPT, D), jnp.float32)

    degp = _sc_degree(dst, ones_k, z1)
    g, dis = _tc_mm1(xp, W1, degp.reshape(NC, NP, 1))
    p = _sc_agg_rows(g, src, dst, z2)
    u = _tc_mm2(p, g, dis, b1.reshape(1, D), W2)
    b2v = jnp.broadcast_to(b2, (16,))
    out = _sc_agg_scalar(u.reshape(NP), dis.reshape(NP), src, dst, z1, b2v)
    return out[:N]


# speedup vs baseline: 19.7165x; 1.4460x over previous
import functools

import jax
import jax.numpy as jnp
from jax import lax
from jax.experimental import pallas as pl
from jax.experimental.pallas import tpu as pltpu
from jax.experimental.pallas import tpu_sc as plsc

N = 10000
E = 320000
D = 128
NP = 10240
NC = 2
NS = 16
NW = NC * NS
KE = 128
CE = 80
EP = NW * CE * KE
WC = 16
TPT = NP // NS
RB = 1024
GRID = NP // RB

_mesh = plsc.VectorSubcoreMesh(core_axis_name="c", subcore_axis_name="s")


@functools.partial(
    pl.kernel,
    out_type=jax.ShapeDtypeStruct((NC, NP), jnp.float32),
    mesh=_mesh,
    scratch_types=[
        pltpu.VMEM((CE, KE), jnp.int32),
        pltpu.VMEM((KE,), jnp.float32),
        pltpu.VMEM_SHARED((NP,), jnp.float32),
    ],
)
def _sc_degree(dst_hbm, ones_hbm, z1_hbm, deg_hbm, dstv, onesv, acc):
    cid = lax.axis_index("c")
    sid = lax.axis_index("s")
    wid = cid * NS + sid
    pltpu.sync_copy(dst_hbm.at[wid], dstv)
    pltpu.sync_copy(ones_hbm, onesv)
    pltpu.sync_copy(z1_hbm, acc.at[pl.ds(sid * TPT, TPT)])
    plsc.subcore_barrier()

    @pl.loop(0, CE)
    def _(c):
        pltpu.sync_copy(onesv, acc.at[dstv.at[c]], add=True)

    plsc.subcore_barrier()
    pltpu.sync_copy(acc.at[pl.ds(sid * TPT, TPT)],
                    deg_hbm.at[cid, pl.ds(sid * TPT, TPT)])


@functools.partial(
    pl.kernel,
    out_type=jax.ShapeDtypeStruct((NC, NP, D), jnp.float32),
    mesh=_mesh,
    scratch_types=[
        pltpu.VMEM((WC, KE), jnp.int32),
        pltpu.VMEM((WC, KE), jnp.int32),
        pltpu.VMEM((KE, D), jnp.float32),
        pltpu.VMEM((KE, D), jnp.float32),
        pltpu.VMEM_SHARED((NP, D), jnp.float32),
        pltpu.SemaphoreType.DMA,
        pltpu.SemaphoreType.DMA,
    ],
)
def _sc_agg_rows(g_hbm, src_hbm, dst_hbm, z2_hbm, out_hbm,
                 srcw, dstw, buf_a, buf_b, acc, sem_a, sem_b):
    cid = lax.axis_index("c")
    sid = lax.axis_index("s")
    wid = cid * NS + sid
    pltpu.sync_copy(z2_hbm, acc.at[pl.ds(sid * TPT, TPT)])
    plsc.subcore_barrier()

    for w in range(CE // WC):
        pltpu.sync_copy(src_hbm.at[wid, pl.ds(w * WC, WC)], srcw)
        pltpu.sync_copy(dst_hbm.at[wid, pl.ds(w * WC, WC)], dstw)
        pltpu.async_copy(g_hbm.at[srcw.at[0]], buf_a, sem_a)

        @pl.loop(0, WC - 2, step=2)
        def _(c):
            pltpu.async_copy(g_hbm.at[srcw.at[c + 1]], buf_b, sem_b)
            pltpu.make_async_copy(g_hbm.at[srcw.at[0]], buf_a, sem_a).wait()
            pltpu.sync_copy(buf_a, acc.at[dstw.at[c]], add=True)
            pltpu.async_copy(g_hbm.at[srcw.at[c + 2]], buf_a, sem_a)
            pltpu.make_async_copy(g_hbm.at[srcw.at[0]], buf_b, sem_b).wait()
            pltpu.sync_copy(buf_b, acc.at[dstw.at[c + 1]], add=True)

        pltpu.async_copy(g_hbm.at[srcw.at[WC - 1]], buf_b, sem_b)
        pltpu.make_async_copy(g_hbm.at[srcw.at[0]], buf_a, sem_a).wait()
        pltpu.sync_copy(buf_a, acc.at[dstw.at[WC - 2]], add=True)
        pltpu.make_async_copy(g_hbm.at[srcw.at[0]], buf_b, sem_b).wait()
        pltpu.sync_copy(buf_b, acc.at[dstw.at[WC - 1]], add=True)

    plsc.subcore_barrier()
    pltpu.sync_copy(acc.at[pl.ds(sid * TPT, TPT)],
                    out_hbm.at[cid, pl.ds(sid * TPT, TPT)])


@functools.partial(
    pl.kernel,
    out_type=jax.ShapeDtypeStruct((NP,), jnp.float32),
    mesh=_mesh,
    scratch_types=[
        pltpu.VMEM((CE, KE), jnp.int32),
        pltpu.VMEM((CE, KE), jnp.int32),
        pltpu.VMEM((KE,), jnp.float32),
        pltpu.VMEM((KE,), jnp.float32),
        pltpu.VMEM((TPT,), jnp.float32),
        pltpu.VMEM((TPT,), jnp.float32),
        pltpu.VMEM((TPT,), jnp.float32),
        pltpu.VMEM((TPT,), jnp.float32),
        pltpu.VMEM((16,), jnp.float32),
        pltpu.VMEM_SHARED((NP,), jnp.float32),
        pltpu.VMEM_SHARED((NP,), jnp.float32),
        pltpu.SemaphoreType.DMA,
        pltpu.SemaphoreType.DMA,
    ],
)
def _sc_agg_scalar(u_hbm, dis_hbm, src_hbm, dst_hbm, z1_hbm, b2_hbm, out_hbm,
                   srcv, dstv, vals_a, vals_b, accv, disv, uv, outv, b2v,
                   acc, u_s, sem_a, sem_b):
    cid = lax.axis_index("c")
    sid = lax.axis_index("s")
    base = sid * TPT
    pltpu.sync_copy(b2_hbm, b2v)
    pltpu.sync_copy(z1_hbm, acc.at[pl.ds(base, TPT)])
    pltpu.sync_copy(u_hbm.at[pl.ds(base, TPT)], uv)
    pltpu.sync_copy(uv, u_s.at[pl.ds(base, TPT)])
    plsc.subcore_barrier()

    for si in (sid, NS + sid):
        pltpu.sync_copy(src_hbm.at[si], srcv)
        pltpu.sync_copy(dst_hbm.at[si], dstv)
        pltpu.async_copy(u_s.at[srcv.at[0]], vals_a, sem_a)

        @pl.loop(0, CE - 2, step=2)
        def _(c):
            pltpu.async_copy(u_s.at[srcv.at[c + 1]], vals_b, sem_b)
            pltpu.make_async_copy(u_s.at[srcv.at[0]], vals_a, sem_a).wait()
            pltpu.sync_copy(vals_a, acc.at[dstv.at[c]], add=True)
            pltpu.async_copy(u_s.at[srcv.at[c + 2]], vals_a, sem_a)
            pltpu.make_async_copy(u_s.at[srcv.at[0]], vals_b, sem_b).wait()
            pltpu.sync_copy(vals_b, acc.at[dstv.at[c + 1]], add=True)

        pltpu.async_copy(u_s.at[srcv.at[CE - 1]], vals_b, sem_b)
        pltpu.make_async_copy(u_s.at[srcv.at[0]], vals_a, sem_a).wait()
        pltpu.sync_copy(vals_a, acc.at[dstv.at[CE - 2]], add=True)
        pltpu.make_async_copy(u_s.at[srcv.at[0]], vals_b, sem_b).wait()
        pltpu.sync_copy(vals_b, acc.at[dstv.at[CE - 1]], add=True)

    plsc.subcore_barrier()
    pltpu.sync_copy(acc.at[pl.ds(base, TPT)], accv)
    pltpu.sync_copy(dis_hbm.at[pl.ds(base, TPT)], disv)
    b2 = b2v[...]

    @pl.loop(0, TPT // 16)
    def _(i):
        a = accv[pl.ds(i * 16, 16)]
        uu = uv[pl.ds(i * 16, 16)]
        dd = disv[pl.ds(i * 16, 16)]
        outv[pl.ds(i * 16, 16)] = dd * (a + uu) + b2

    @pl.when(cid == 0)
    def _():
        pltpu.sync_copy(outv, out_hbm.at[pl.ds(base, TPT)])


def _tc_mm1(x, w1, degp):
    def body(xr, wr, dr, gr, disr):
        h = jnp.dot(xr[...], wr[...], preferred_element_type=jnp.float32)
        dis = lax.rsqrt(dr[0] + dr[1] + 1.0)
        disr[...] = dis
        gr[...] = h * dis

    return pl.pallas_call(
        body,
        grid=(GRID,),
        in_specs=[
            pl.BlockSpec((RB, D), lambda i: (i, 0)),
            pl.BlockSpec((D, D), lambda i: (0, 0)),
            pl.BlockSpec((NC, RB, 1), lambda i: (0, i, 0)),
        ],
        out_specs=[
            pl.BlockSpec((RB, D), lambda i: (i, 0)),
            pl.BlockSpec((RB, 1), lambda i: (i, 0)),
        ],
        out_shape=[
            jax.ShapeDtypeStruct((NP, D), jnp.float32),
            jax.ShapeDtypeStruct((NP, 1), jnp.float32),
        ],
    )(x, w1, degp)


def _tc_mm2(p, g, dis, b1, w2):
    def body(pr, gr, disr, br, wr, ur):
        s = (pr[0] + pr[1] + gr[...]) * disr[...] + br[...]
        h1 = jnp.maximum(s, 0.0)
        t = jnp.dot(h1, wr[...], preferred_element_type=jnp.float32)
        ur[...] = t * disr[...]

    return pl.pallas_call(
        body,
        grid=(GRID,),
        in_specs=[
            pl.BlockSpec((NC, RB, D), lambda i: (0, i, 0)),
            pl.BlockSpec((RB, D), lambda i: (i, 0)),
            pl.BlockSpec((RB, 1), lambda i: (i, 0)),
            pl.BlockSpec((1, D), lambda i: (0, 0)),
            pl.BlockSpec((D, 1), lambda i: (0, 0)),
        ],
        out_specs=pl.BlockSpec((RB, 1), lambda i: (i, 0)),
        out_shape=jax.ShapeDtypeStruct((NP, 1), jnp.float32),
    )(p, g, dis, b1, w2)


@jax.jit
def kernel(x, edge_index, W1, b1, W2, b2):
    pad = jnp.zeros((EP - E,), jnp.int32)
    src = jnp.concatenate([edge_index[0], pad]).reshape(NW, CE, KE)
    dst = jnp.concatenate([edge_index[1], pad + N]).reshape(NW, CE, KE)
    xp = jnp.pad(x, ((0, NP - N), (0, 0)))
    ones_k = jnp.ones((KE,), jnp.float32)
    z1 = jnp.zeros((TPT,), jnp.float32)
    z2 = jnp.zeros((TPT, D), jnp.float32)

    degp = _sc_degree(dst, ones_k, z1)
    g, dis = _tc_mm1(xp, W1, degp.reshape(NC, NP, 1))
    p = _sc_agg_rows(g, src, dst, z2)
    u = _tc_mm2(p, g, dis, b1.reshape(1, D), W2)
    b2v = jnp.broadcast_to(b2, (16,))
    out = _sc_agg_scalar(u.reshape(NP), dis.reshape(NP), src, dst, z1, b2v)
    return out[:N]


# speedup vs baseline: 33.1642x; 1.6821x over previous
import functools

import jax
import jax.numpy as jnp
from jax import lax
from jax.experimental import pallas as pl
from jax.experimental.pallas import tpu as pltpu
from jax.experimental.pallas import tpu_sc as plsc

N = 10000
E = 320000
D = 128
NP = 10240
NC = 2
NS = 16
NW = NC * NS
KE = 128
CE = 80
EP = NW * CE * KE
WC = 16
TPT = NP // NS
RB = 1024
GRID = NP // RB

_mesh = plsc.VectorSubcoreMesh(core_axis_name="c", subcore_axis_name="s")


@functools.partial(
    pl.kernel,
    out_type=jax.ShapeDtypeStruct((NC, NP), jnp.float32),
    mesh=_mesh,
    scratch_types=[
        pltpu.VMEM((CE, KE), jnp.int32),
        pltpu.VMEM((KE,), jnp.float32),
        pltpu.VMEM_SHARED((NP,), jnp.float32),
    ],
)
def _sc_degree(dst_hbm, ones_hbm, z1_hbm, deg_hbm, dstv, onesv, acc):
    cid = lax.axis_index("c")
    sid = lax.axis_index("s")
    wid = cid * NS + sid
    pltpu.sync_copy(dst_hbm.at[wid], dstv)
    pltpu.sync_copy(ones_hbm, onesv)
    pltpu.sync_copy(z1_hbm, acc.at[pl.ds(sid * TPT, TPT)])
    plsc.subcore_barrier()

    @pl.loop(0, CE)
    def _(c):
        pltpu.sync_copy(onesv, acc.at[dstv.at[c]], add=True)

    plsc.subcore_barrier()
    pltpu.sync_copy(acc.at[pl.ds(sid * TPT, TPT)],
                    deg_hbm.at[cid, pl.ds(sid * TPT, TPT)])


DH = D // NC


@functools.partial(
    pl.kernel,
    out_type=jax.ShapeDtypeStruct((NC, NP, DH), jnp.float32),
    mesh=_mesh,
    scratch_types=[
        pltpu.VMEM((WC, KE), jnp.int32),
        pltpu.VMEM((WC, KE), jnp.int32),
        pltpu.VMEM((KE, DH), jnp.float32),
        pltpu.VMEM((KE, DH), jnp.float32),
        pltpu.VMEM_SHARED((NP, DH), jnp.float32),
        pltpu.VMEM_SHARED((NP, DH), jnp.float32),
        pltpu.SemaphoreType.DMA,
        pltpu.SemaphoreType.DMA,
    ],
    compiler_params=pltpu.CompilerParams(use_tc_tiling_on_sc=False),
)
def _sc_agg_rows(g01_hbm, src_hbm, dst_hbm, z2_hbm, out_hbm,
                 srcw, dstw, b0, b1, gs, acc, s0, s1):
    cid = lax.axis_index("c")
    sid = lax.axis_index("s")
    RING = 2
    bufs = (b0, b1)
    sems = (s0, s1)
    pltpu.sync_copy(g01_hbm.at[cid, pl.ds(sid * TPT, TPT)],
                    gs.at[pl.ds(sid * TPT, TPT)])
    pltpu.sync_copy(z2_hbm, acc.at[pl.ds(sid * TPT, TPT)])
    plsc.subcore_barrier()

    def start(ci, slot):
        pltpu.async_copy(gs.at[srcw.at[ci]], bufs[slot], sems[slot])

    def finish(ci, slot):
        pltpu.make_async_copy(gs.at[srcw.at[0]], bufs[slot], sems[slot]).wait()
        pltpu.sync_copy(bufs[slot], acc.at[dstw.at[ci]], add=True)

    for si in (sid, NS + sid):
        for w in range(CE // WC):
            pltpu.sync_copy(src_hbm.at[si, pl.ds(w * WC, WC)], srcw)
            pltpu.sync_copy(dst_hbm.at[si, pl.ds(w * WC, WC)], dstw)
            for j in range(RING - 1):
                start(j, j)

            @pl.loop(0, WC - RING, step=RING)
            def _(c):
                for j in range(RING):
                    start(c + j + RING - 1, (j + RING - 1) % RING)
                    finish(c + j, j)

            start(WC - 1, (WC - 1) % RING)
            for j in range(RING):
                finish(WC - RING + j, j)

    plsc.subcore_barrier()
    pltpu.sync_copy(acc.at[pl.ds(sid * TPT, TPT)],
                    out_hbm.at[cid, pl.ds(sid * TPT, TPT)])


@functools.partial(
    pl.kernel,
    out_type=jax.ShapeDtypeStruct((NP,), jnp.float32),
    mesh=_mesh,
    scratch_types=[
        pltpu.VMEM((CE, KE), jnp.int32),
        pltpu.VMEM((CE, KE), jnp.int32),
        pltpu.VMEM((KE,), jnp.float32),
        pltpu.VMEM((KE,), jnp.float32),
        pltpu.VMEM((TPT,), jnp.float32),
        pltpu.VMEM((TPT,), jnp.float32),
        pltpu.VMEM((TPT,), jnp.float32),
        pltpu.VMEM((TPT,), jnp.float32),
        pltpu.VMEM((16,), jnp.float32),
        pltpu.VMEM_SHARED((NP,), jnp.float32),
        pltpu.VMEM_SHARED((NP,), jnp.float32),
        pltpu.SemaphoreType.DMA,
        pltpu.SemaphoreType.DMA,
    ],
)
def _sc_agg_scalar(u_hbm, dis_hbm, src_hbm, dst_hbm, z1_hbm, b2_hbm, out_hbm,
                   srcv, dstv, vals_a, vals_b, accv, disv, uv, outv, b2v,
                   acc, u_s, sem_a, sem_b):
    cid = lax.axis_index("c")
    sid = lax.axis_index("s")
    base = sid * TPT
    pltpu.sync_copy(b2_hbm, b2v)
    pltpu.sync_copy(z1_hbm, acc.at[pl.ds(base, TPT)])
    pltpu.sync_copy(u_hbm.at[pl.ds(base, TPT)], uv)
    pltpu.sync_copy(uv, u_s.at[pl.ds(base, TPT)])
    plsc.subcore_barrier()

    for si in (sid, NS + sid):
        pltpu.sync_copy(src_hbm.at[si], srcv)
        pltpu.sync_copy(dst_hbm.at[si], dstv)
        pltpu.async_copy(u_s.at[srcv.at[0]], vals_a, sem_a)

        @pl.loop(0, CE - 2, step=2)
        def _(c):
            pltpu.async_copy(u_s.at[srcv.at[c + 1]], vals_b, sem_b)
            pltpu.make_async_copy(u_s.at[srcv.at[0]], vals_a, sem_a).wait()
            pltpu.sync_copy(vals_a, acc.at[dstv.at[c]], add=True)
            pltpu.async_copy(u_s.at[srcv.at[c + 2]], vals_a, sem_a)
            pltpu.make_async_copy(u_s.at[srcv.at[0]], vals_b, sem_b).wait()
            pltpu.sync_copy(vals_b, acc.at[dstv.at[c + 1]], add=True)

        pltpu.async_copy(u_s.at[srcv.at[CE - 1]], vals_b, sem_b)
        pltpu.make_async_copy(u_s.at[srcv.at[0]], vals_a, sem_a).wait()
        pltpu.sync_copy(vals_a, acc.at[dstv.at[CE - 2]], add=True)
        pltpu.make_async_copy(u_s.at[srcv.at[0]], vals_b, sem_b).wait()
        pltpu.sync_copy(vals_b, acc.at[dstv.at[CE - 1]], add=True)

    plsc.subcore_barrier()
    pltpu.sync_copy(acc.at[pl.ds(base, TPT)], accv)
    pltpu.sync_copy(dis_hbm.at[pl.ds(base, TPT)], disv)
    b2 = b2v[...]

    @pl.loop(0, TPT // 16)
    def _(i):
        a = accv[pl.ds(i * 16, 16)]
        uu = uv[pl.ds(i * 16, 16)]
        dd = disv[pl.ds(i * 16, 16)]
        outv[pl.ds(i * 16, 16)] = dd * (a + uu) + b2

    @pl.when(cid == 0)
    def _():
        pltpu.sync_copy(outv, out_hbm.at[pl.ds(base, TPT)])


def _tc_mm1(x, w1, degp):
    def body(xr, wr, dr, gr, disr):
        h = jnp.dot(xr[...], wr[...], preferred_element_type=jnp.float32)
        dis = lax.rsqrt(dr[0] + dr[1] + 1.0)
        disr[...] = dis
        gr[0] = h[:, :DH] * dis
        gr[1] = h[:, DH:] * dis

    return pl.pallas_call(
        body,
        grid=(GRID,),
        in_specs=[
            pl.BlockSpec((RB, D), lambda i: (i, 0)),
            pl.BlockSpec((D, D), lambda i: (0, 0)),
            pl.BlockSpec((NC, RB, 1), lambda i: (0, i, 0)),
        ],
        out_specs=[
            pl.BlockSpec((NC, RB, DH), lambda i: (0, i, 0)),
            pl.BlockSpec((RB, 1), lambda i: (i, 0)),
        ],
        out_shape=[
            jax.ShapeDtypeStruct((NC, NP, DH), jnp.float32),
            jax.ShapeDtypeStruct((NP, 1), jnp.float32),
        ],
    )(x, w1, degp)


def _tc_mm2(p, g01, dis, b1, w2):
    def body(pr, gr, disr, br, wr, ur):
        t = jnp.zeros((RB, 1), jnp.float32)
        for c in range(NC):
            s = (pr[c] + gr[c]) * disr[...] + br[c]
            h1 = jnp.maximum(s, 0.0)
            t = t + jnp.dot(h1, wr[c], preferred_element_type=jnp.float32)
        ur[...] = t * disr[...]

    return pl.pallas_call(
        body,
        grid=(GRID,),
        in_specs=[
            pl.BlockSpec((NC, RB, DH), lambda i: (0, i, 0)),
            pl.BlockSpec((NC, RB, DH), lambda i: (0, i, 0)),
            pl.BlockSpec((RB, 1), lambda i: (i, 0)),
            pl.BlockSpec((NC, DH), lambda i: (0, 0)),
            pl.BlockSpec((NC, DH, 1), lambda i: (0, 0, 0)),
        ],
        out_specs=pl.BlockSpec((RB, 1), lambda i: (i, 0)),
        out_shape=jax.ShapeDtypeStruct((NP, 1), jnp.float32),
    )(p, g01, dis, b1, w2)


@jax.jit
def kernel(x, edge_index, W1, b1, W2, b2):
    pad = jnp.zeros((EP - E,), jnp.int32)
    src = jnp.concatenate([edge_index[0], pad]).reshape(NW, CE, KE)
    dst = jnp.concatenate([edge_index[1], pad + N]).reshape(NW, CE, KE)
    xp = jnp.pad(x, ((0, NP - N), (0, 0)))
    ones_k = jnp.ones((KE,), jnp.float32)
    z1 = jnp.zeros((TPT,), jnp.float32)
    z2 = jnp.zeros((TPT, DH), jnp.float32)

    degp = _sc_degree(dst, ones_k, z1)
    g01, dis = _tc_mm1(xp, W1, degp.reshape(NC, NP, 1))
    p = _sc_agg_rows(g01, src, dst, z2)
    u = _tc_mm2(p, g01, dis, b1.reshape(NC, DH), W2.reshape(NC, DH, 1))
    b2v = jnp.broadcast_to(b2, (16,))
    out = _sc_agg_scalar(u.reshape(NP), dis.reshape(NP), src, dst, z1, b2v)
    return out[:N]
